# Initial kernel scaffold; baseline (speedup 1.0000x reference)
import jax
import jax.numpy as jnp
from jax.experimental import pallas as pl


def kernel(x, edge_index, W1_l, b1_l, W1_r, W2_l, b2_l, W2_r, Wfc, bfc):
    raise NotImplementedError("write your pallas kernel here")


# speedup vs baseline: 7.1777x; 7.1777x over previous
import functools

import jax
import jax.numpy as jnp
from jax import lax
from jax.experimental import pallas as pl
from jax.experimental.pallas import tpu as pltpu
from jax.experimental.pallas import tpu_sc as plsc

N = 10000
E = 320000
D_IN = 128
D_HID = 128
D_HID2 = 64

CHUNK = 128
NCHUNKS = E // CHUNK
NCORES = 2
NSUB = 16
CH_PER_CORE = NCHUNKS // NCORES
JMAX = -(-CH_PER_CORE // NSUB)
ROWS_Q = 624
TAIL = N - NSUB * ROWS_Q


def _tc_pre_body(x_ref, wl_ref, bl_ref, wr_ref, p_ref, r_ref):
    x = x_ref[...]
    dn = (((1,), (1,)), ((), ()))
    p_ref[...] = lax.dot_general(x, wl_ref[...], dn,
                                 preferred_element_type=jnp.float32)
    r_ref[...] = lax.dot_general(x, wr_ref[...], dn,
                                 preferred_element_type=jnp.float32) + bl_ref[...]


def _tc_pre(x, wl, bl, wr):
    return pl.pallas_call(
        _tc_pre_body,
        out_shape=(jax.ShapeDtypeStruct((N, D_HID), jnp.float32),
                   jax.ShapeDtypeStruct((N, D_HID), jnp.float32)),
    )(x, wl, bl, wr)


def _tc_mid_body(acc_ref, deg_ref, r1_ref, w2l_ref, b2l_ref, w2r_ref,
                 p2_ref, r2_ref):
    dsum = deg_ref[0] + deg_ref[1]
    recip = 1.0 / jnp.maximum(dsum, 1.0)
    mean = (acc_ref[0] + acc_ref[1]) * recip
    h = jnp.maximum(mean + r1_ref[...], 0.0)
    dn = (((1,), (1,)), ((), ()))
    p2_ref[...] = lax.dot_general(h, w2l_ref[...], dn,
                                  preferred_element_type=jnp.float32)
    r2_ref[...] = lax.dot_general(h, w2r_ref[...], dn,
                                  preferred_element_type=jnp.float32) + b2l_ref[...]


def _tc_mid(acc, deg, r1, w2l, b2l, w2r):
    return pl.pallas_call(
        _tc_mid_body,
        out_shape=(jax.ShapeDtypeStruct((N, D_HID2), jnp.float32),
                   jax.ShapeDtypeStruct((N, D_HID2), jnp.float32)),
    )(acc, deg, r1, w2l, b2l, w2r)


def _tc_post_body(acc_ref, deg_ref, r2_ref, wfc_ref, bfc_ref, out_ref):
    dsum = deg_ref[0] + deg_ref[1]
    recip = 1.0 / jnp.maximum(dsum, 1.0)
    mean = (acc_ref[0] + acc_ref[1]) * recip
    h = jnp.maximum(mean + r2_ref[...], 0.0)
    logits = jnp.sum(h * wfc_ref[...], axis=1, keepdims=True) + bfc_ref[0, 0]
    out_ref[...] = jax.nn.sigmoid(logits)


def _tc_post(acc, deg, r2, wfc, bfc):
    return pl.pallas_call(
        _tc_post_body,
        out_shape=jax.ShapeDtypeStruct((N, 1), jnp.float32),
    )(acc, deg, r2, wfc, bfc)


def _make_sc_agg(D, with_deg):
    mesh = plsc.VectorSubcoreMesh(core_axis_name="c", subcore_axis_name="s")
    out_type = [jax.ShapeDtypeStruct((NCORES, N, D), jnp.float32)]
    scratch = [
        pltpu.VMEM((CHUNK,), jnp.int32),
        pltpu.VMEM((CHUNK,), jnp.int32),
        pltpu.VMEM((CHUNK, D), jnp.float32),
        pltpu.VMEM_SHARED((N, D), jnp.float32),
        pltpu.SemaphoreType.DMA,
    ]
    if with_deg:
        out_type.append(jax.ShapeDtypeStruct((NCORES, N), jnp.float32))
        scratch += [
            pltpu.VMEM((CHUNK,), jnp.float32),
            pltpu.VMEM((N,), jnp.float32),
            pltpu.VMEM_SHARED((N,), jnp.float32),
        ]

    @functools.partial(pl.kernel, mesh=mesh, out_type=out_type,
                       scratch_types=scratch,
                       compiler_params=pltpu.CompilerParams(
                           use_tc_tiling_on_sc=False))
    def k(p_hbm, src_hbm, dst_hbm, acc_out, *rest):
        if with_deg:
            deg_out, src_v, dst_v, rows_v, acc_sh, sem, ones_v, zn_v, deg_sh = rest
        else:
            src_v, dst_v, rows_v, acc_sh, sem = rest
            deg_out = zn_v = ones_v = deg_sh = None

        c = lax.axis_index("c")
        s = lax.axis_index("s")
        zero16 = jnp.zeros((16,), jnp.float32)

        def zrow(r, _):
            for k8 in range(D // 16):
                rows_v[r, pl.ds(k8 * 16, 16)] = zero16
            return 0
        lax.fori_loop(0, CHUNK, zrow, 0)

        base = s * ROWS_Q
        for kk in range(ROWS_Q // CHUNK):
            pltpu.sync_copy(rows_v, acc_sh.at[pl.ds(base + kk * CHUNK, CHUNK)])
        rem = ROWS_Q % CHUNK
        if rem:
            pltpu.sync_copy(rows_v.at[pl.ds(0, rem)],
                            acc_sh.at[pl.ds(base + (ROWS_Q // CHUNK) * CHUNK, rem)])

        @pl.when(s == NSUB - 1)
        def _():
            pltpu.sync_copy(rows_v.at[pl.ds(0, TAIL)],
                            acc_sh.at[pl.ds(NSUB * ROWS_Q, TAIL)])

        if with_deg:
            one16 = jnp.ones((16,), jnp.float32)
            for k8 in range(CHUNK // 16):
                ones_v[pl.ds(k8 * 16, 16)] = one16

            def zdeg(i, _):
                zn_v[pl.ds(i * 16, 16)] = zero16
                return 0
            lax.fori_loop(0, N // 16, zdeg, 0)

            @pl.when(s == 0)
            def _():
                pltpu.sync_copy(zn_v, deg_sh)

        plsc.subcore_barrier()

        cbase = c * CH_PER_CORE

        def body(j, _):
            rel = s + NSUB * j

            @pl.when(rel < CH_PER_CORE)
            def _():
                cid = cbase + rel
                pltpu.sync_copy(src_hbm.at[cid], src_v)
                pltpu.sync_copy(dst_hbm.at[cid], dst_v)
                pltpu.async_copy(p_hbm.at[src_v], rows_v, sem).wait()
                pltpu.sync_copy(rows_v, acc_sh.at[dst_v], add=True)
                if with_deg:
                    pltpu.sync_copy(ones_v, deg_sh.at[dst_v], add=True)
            return 0
        lax.fori_loop(0, JMAX, body, 0)

        plsc.subcore_barrier()

        pltpu.sync_copy(acc_sh.at[pl.ds(base, ROWS_Q)],
                        acc_out.at[c, pl.ds(base, ROWS_Q)])

        @pl.when(s == NSUB - 1)
        def _():
            pltpu.sync_copy(acc_sh.at[pl.ds(NSUB * ROWS_Q, TAIL)],
                            acc_out.at[c, pl.ds(NSUB * ROWS_Q, TAIL)])
        if with_deg:
            @pl.when(s == 0)
            def _():
                pltpu.sync_copy(deg_sh, deg_out.at[c])

    return k


_sc_agg_l1 = _make_sc_agg(D_HID, with_deg=True)
_sc_agg_l2 = _make_sc_agg(D_HID2, with_deg=False)


def kernel(x, edge_index, W1_l, b1_l, W1_r, W2_l, b2_l, W2_r, Wfc, bfc):
    src2d = edge_index[0].reshape(NCHUNKS, CHUNK)
    dst2d = edge_index[1].reshape(NCHUNKS, CHUNK)

    p1, r1 = _tc_pre(x, W1_l, b1_l.reshape(1, D_HID), W1_r)
    acc1, deg = _sc_agg_l1(p1, src2d, dst2d)
    deg3 = deg.reshape(NCORES, N, 1)
    p2, r2 = _tc_mid(acc1, deg3, r1, W2_l, b2_l.reshape(1, D_HID2), W2_r)
    acc2 = _sc_agg_l2(p2, src2d, dst2d)
    if isinstance(acc2, (list, tuple)):
        acc2 = acc2[0]
    out = _tc_post(acc2, deg3, r2, Wfc, bfc.reshape(1, 1))
    return out


# speedup vs baseline: 15.2865x; 2.1297x over previous
import functools

import jax
import jax.numpy as jnp
from jax import lax
from jax.experimental import pallas as pl
from jax.experimental.pallas import tpu as pltpu
from jax.experimental.pallas import tpu_sc as plsc

N = 10000
E = 320000
D_HID = 128
D_HID2 = 64
DH = D_HID // 2

CHUNK = 128
NCHUNKS = E // CHUNK
NCORES = 2
NSUB = 16
NTILES = NCORES * NSUB
NBUF = 4

C1_BASE = NCHUNKS // NSUB
C1_EXTRA = NCHUNKS % NSUB
C1_MAX = C1_BASE + 1
J1PAD = 160
C2_BASE = NCHUNKS // NTILES
C2_EXTRA = NCHUNKS % NTILES
C2_MAX = C2_BASE + 1
J2PAD = 80

ROWS_Q = 624
TAIL = N - NSUB * ROWS_Q


def _tc_pre_body(x_ref, wla_ref, wlb_ref, bl_ref, wr_ref,
                 pa_ref, pb_ref, r_ref):
    x = x_ref[...]
    dn = (((1,), (1,)), ((), ()))
    pa_ref[...] = lax.dot_general(x, wla_ref[...], dn,
                                  preferred_element_type=jnp.float32)
    pb_ref[...] = lax.dot_general(x, wlb_ref[...], dn,
                                  preferred_element_type=jnp.float32)
    r_ref[...] = lax.dot_general(x, wr_ref[...], dn,
                                 preferred_element_type=jnp.float32) + bl_ref[...]


def _tc_pre(x, wla, wlb, bl, wr):
    return pl.pallas_call(
        _tc_pre_body,
        out_shape=(jax.ShapeDtypeStruct((N, DH), jnp.float32),
                   jax.ShapeDtypeStruct((N, DH), jnp.float32),
                   jax.ShapeDtypeStruct((N, D_HID), jnp.float32)),
    )(x, wla, wlb, bl, wr)


def _tc_mid_body(acca_ref, accb_ref, deg_ref, r1_ref,
                 w2la_ref, w2lb_ref, b2l_ref, w2ra_ref, w2rb_ref,
                 p2_ref, r2_ref):
    recip = 1.0 / jnp.maximum(deg_ref[...], 1.0)
    ha = jnp.maximum(acca_ref[...] * recip + r1_ref[:, :DH], 0.0)
    hb = jnp.maximum(accb_ref[...] * recip + r1_ref[:, DH:], 0.0)
    dn = (((1,), (1,)), ((), ()))
    p2_ref[...] = (
        lax.dot_general(ha, w2la_ref[...], dn, preferred_element_type=jnp.float32)
        + lax.dot_general(hb, w2lb_ref[...], dn, preferred_element_type=jnp.float32))
    r2_ref[...] = (
        lax.dot_general(ha, w2ra_ref[...], dn, preferred_element_type=jnp.float32)
        + lax.dot_general(hb, w2rb_ref[...], dn, preferred_element_type=jnp.float32)
        + b2l_ref[...])


def _tc_mid(acca, accb, deg, r1, w2la, w2lb, b2l, w2ra, w2rb):
    return pl.pallas_call(
        _tc_mid_body,
        out_shape=(jax.ShapeDtypeStruct((N, D_HID2), jnp.float32),
                   jax.ShapeDtypeStruct((N, D_HID2), jnp.float32)),
    )(acca, accb, deg, r1, w2la, w2lb, b2l, w2ra, w2rb)


def _tc_post_body(acc_ref, deg_ref, r2_ref, wfc_ref, bfc_ref, out_ref):
    recip = 1.0 / jnp.maximum(deg_ref[...], 1.0)
    mean = (acc_ref[0] + acc_ref[1]) * recip
    h = jnp.maximum(mean + r2_ref[...], 0.0)
    logits = jnp.sum(h * wfc_ref[...], axis=1, keepdims=True) + bfc_ref[0, 0]
    out_ref[...] = jax.nn.sigmoid(logits)


def _tc_post(acc, deg, r2, wfc, bfc):
    return pl.pallas_call(
        _tc_post_body,
        out_shape=jax.ShapeDtypeStruct((N, 1), jnp.float32),
    )(acc, deg, r2, wfc, bfc)


_SC_PARAMS = pltpu.CompilerParams(use_tc_tiling_on_sc=False)
_MESH = dict(core_axis_name="c", subcore_axis_name="s")


def _zero_rows_buf(buf, d):
    zero16 = jnp.zeros((16,), jnp.float32)

    def zrow(r, _):
        for k8 in range(d // 16):
            buf[r, pl.ds(k8 * 16, 16)] = zero16
        return 0
    lax.fori_loop(0, CHUNK, zrow, 0)


def _zero_acc_slice(acc_sh, s, zbuf):
    base = s * ROWS_Q
    for kk in range(ROWS_Q // CHUNK):
        pltpu.sync_copy(zbuf, acc_sh.at[pl.ds(base + kk * CHUNK, CHUNK)])
    rem = ROWS_Q % CHUNK
    if rem:
        pltpu.sync_copy(zbuf.at[pl.ds(0, rem)],
                        acc_sh.at[pl.ds(base + (ROWS_Q // CHUNK) * CHUNK, rem)])

    @pl.when(s == NSUB - 1)
    def _():
        pltpu.sync_copy(zbuf.at[pl.ds(0, TAIL)],
                        acc_sh.at[pl.ds(NSUB * ROWS_Q, TAIL)])


def _copy_acc_out(acc_sh, s, dst):
    base = s * ROWS_Q
    pltpu.sync_copy(acc_sh.at[pl.ds(base, ROWS_Q)], dst.at[pl.ds(base, ROWS_Q)])

    @pl.when(s == NSUB - 1)
    def _():
        pltpu.sync_copy(acc_sh.at[pl.ds(NSUB * ROWS_Q, TAIL)],
                        dst.at[pl.ds(NSUB * ROWS_Q, TAIL)])


@functools.partial(
    pl.kernel,
    mesh=plsc.VectorSubcoreMesh(**_MESH),
    out_type=[jax.ShapeDtypeStruct((N, DH), jnp.float32),
              jax.ShapeDtypeStruct((N, DH), jnp.float32),
              jax.ShapeDtypeStruct((N,), jnp.float32)],
    scratch_types=(
        [pltpu.VMEM((C1_MAX, CHUNK), jnp.int32),
         pltpu.VMEM((C1_MAX, CHUNK), jnp.int32)]
        + [pltpu.VMEM((CHUNK, DH), jnp.float32) for _ in range(NBUF)]
        + [pltpu.VMEM_SHARED((N, DH), jnp.float32),
           pltpu.SemaphoreType.DMA((NBUF,)),
           pltpu.SemaphoreType.DMA((NBUF,)),
           pltpu.SemaphoreType.DMA((NBUF,)),
           pltpu.VMEM((CHUNK,), jnp.float32),
           pltpu.VMEM((N,), jnp.float32),
           pltpu.VMEM_SHARED((N,), jnp.float32)]
    ),
    compiler_params=_SC_PARAMS,
)
def _sc_agg1(pa_hbm, pb_hbm, src_hbm, dst_hbm, acca_out, accb_out, deg_out,
             src_iv, dst_iv, r0, r1, r2, r3, acc_sh, gsem, ssem, dsem,
             ones_v, zn_v, deg_sh):
    rows = [r0, r1, r2, r3]
    c = lax.axis_index("c")
    s = lax.axis_index("s")
    nt = jnp.where(s < C1_EXTRA, C1_BASE + 1, C1_BASE)
    t0 = s * C1_BASE + jnp.minimum(s, C1_EXTRA)
    zero16 = jnp.zeros((16,), jnp.float32)

    @pl.when(s < C1_EXTRA)
    def _():
        pltpu.sync_copy(src_hbm.at[pl.ds(t0, C1_MAX)], src_iv)
        pltpu.sync_copy(dst_hbm.at[pl.ds(t0, C1_MAX)], dst_iv)

    @pl.when(s >= C1_EXTRA)
    def _():
        pltpu.sync_copy(src_hbm.at[pl.ds(t0, C1_BASE)],
                        src_iv.at[pl.ds(0, C1_BASE)])
        pltpu.sync_copy(dst_hbm.at[pl.ds(t0, C1_BASE)],
                        dst_iv.at[pl.ds(0, C1_BASE)])

    _zero_rows_buf(rows[0], DH)
    _zero_acc_slice(acc_sh, s, rows[0])

    one16 = jnp.ones((16,), jnp.float32)
    for k8 in range(CHUNK // 16):
        ones_v[pl.ds(k8 * 16, 16)] = one16

    @pl.when(jnp.logical_and(c == 0, s == 0))
    def _():
        def zdeg(i, _):
            zn_v[pl.ds(i * 16, 16)] = zero16
            return 0
        lax.fori_loop(0, N // 16, zdeg, 0)
        pltpu.sync_copy(zn_v, deg_sh)

    plsc.subcore_barrier()

    def gather(j, b):
        @pl.when(c == 0)
        def _():
            pltpu.async_copy(pa_hbm.at[src_iv.at[j]], rows[b], gsem.at[b])

        @pl.when(c == 1)
        def _():
            pltpu.async_copy(pb_hbm.at[src_iv.at[j]], rows[b], gsem.at[b])

    for b in range(NBUF):
        gather(b, b)

    def body(jo, _):
        for b in range(NBUF):
            j = jo * NBUF + b

            @pl.when(j < nt)
            def _():
                pltpu.make_async_copy(pa_hbm.at[src_iv.at[j]], rows[b],
                                      gsem.at[b]).wait()
                sd = pltpu.async_copy(rows[b], acc_sh.at[dst_iv.at[j]],
                                      ssem.at[b], add=True)

                @pl.when(c == 0)
                def _():
                    pltpu.async_copy(ones_v, deg_sh.at[dst_iv.at[j]],
                                     dsem.at[b], add=True).wait()
                sd.wait()

                @pl.when(j + NBUF < nt)
                def _():
                    gather(j + NBUF, b)
        return 0
    lax.fori_loop(0, J1PAD // NBUF, body, 0)

    plsc.subcore_barrier()

    @pl.when(c == 0)
    def _():
        _copy_acc_out(acc_sh, s, acca_out)

        @pl.when(s == 0)
        def _():
            pltpu.sync_copy(deg_sh, deg_out)

    @pl.when(c == 1)
    def _():
        _copy_acc_out(acc_sh, s, accb_out)


@functools.partial(
    pl.kernel,
    mesh=plsc.VectorSubcoreMesh(**_MESH),
    out_type=[jax.ShapeDtypeStruct((NCORES, N, D_HID2), jnp.float32)],
    scratch_types=(
        [pltpu.VMEM((C2_MAX, CHUNK), jnp.int32),
         pltpu.VMEM((C2_MAX, CHUNK), jnp.int32)]
        + [pltpu.VMEM((CHUNK, D_HID2), jnp.float32) for _ in range(NBUF)]
        + [pltpu.VMEM_SHARED((N, D_HID2), jnp.float32),
           pltpu.SemaphoreType.DMA((NBUF,)),
           pltpu.SemaphoreType.DMA((NBUF,))]
    ),
    compiler_params=_SC_PARAMS,
)
def _sc_agg2(p_hbm, src_hbm, dst_hbm, acc_out,
             src_iv, dst_iv, r0, r1, r2, r3, acc_sh, gsem, ssem):
    rows = [r0, r1, r2, r3]
    c = lax.axis_index("c")
    s = lax.axis_index("s")
    w = c * NSUB + s
    nt = jnp.where(w < C2_EXTRA, C2_BASE + 1, C2_BASE)
    t0 = w * C2_BASE + jnp.minimum(w, C2_EXTRA)

    @pl.when(w < C2_EXTRA)
    def _():
        pltpu.sync_copy(src_hbm.at[pl.ds(t0, C2_MAX)], src_iv)
        pltpu.sync_copy(dst_hbm.at[pl.ds(t0, C2_MAX)], dst_iv)

    @pl.when(w >= C2_EXTRA)
    def _():
        pltpu.sync_copy(src_hbm.at[pl.ds(t0, C2_BASE)],
                        src_iv.at[pl.ds(0, C2_BASE)])
        pltpu.sync_copy(dst_hbm.at[pl.ds(t0, C2_BASE)],
                        dst_iv.at[pl.ds(0, C2_BASE)])

    _zero_rows_buf(rows[0], D_HID2)
    _zero_acc_slice(acc_sh, s, rows[0])

    plsc.subcore_barrier()

    def gather(j, b):
        pltpu.async_copy(p_hbm.at[src_iv.at[j]], rows[b], gsem.at[b])

    for b in range(NBUF):
        gather(b, b)

    def body(jo, _):
        for b in range(NBUF):
            j = jo * NBUF + b

            @pl.when(j < nt)
            def _():
                pltpu.make_async_copy(p_hbm.at[src_iv.at[j]], rows[b],
                                      gsem.at[b]).wait()
                pltpu.async_copy(rows[b], acc_sh.at[dst_iv.at[j]],
                                 ssem.at[b], add=True).wait()

                @pl.when(j + NBUF < nt)
                def _():
                    gather(j + NBUF, b)
        return 0
    lax.fori_loop(0, J2PAD // NBUF, body, 0)

    plsc.subcore_barrier()

    _copy_acc_out(acc_sh, s, acc_out.at[c])


def kernel(x, edge_index, W1_l, b1_l, W1_r, W2_l, b2_l, W2_r, Wfc, bfc):
    src2d = edge_index[0].reshape(NCHUNKS, CHUNK)
    dst2d = edge_index[1].reshape(NCHUNKS, CHUNK)

    p1a, p1b, r1 = _tc_pre(x, W1_l[:DH], W1_l[DH:], b1_l.reshape(1, D_HID),
                           W1_r)
    acca, accb, deg = _sc_agg1(p1a, p1b, src2d, dst2d)
    deg2 = deg.reshape(N, 1)
    p2, r2 = _tc_mid(acca, accb, deg2, r1,
                     W2_l[:, :DH], W2_l[:, DH:], b2_l.reshape(1, D_HID2),
                     W2_r[:, :DH], W2_r[:, DH:])
    acc2 = _sc_agg2(p2, src2d, dst2d)
    if isinstance(acc2, (list, tuple)):
        acc2 = acc2[0]
    out = _tc_post(acc2, deg2, r2, Wfc, bfc.reshape(1, 1))
    return out
